# baseline (speedup 1.0000x reference)
import functools

import jax
import jax.numpy as jnp
from jax import lax
from jax.experimental import pallas as pl
from jax.experimental.pallas import tpu as pltpu
from jax.experimental.pallas import tpu_sc as plsc

VOCAB = 1000000
D = 64
SEQ = 200
S0 = 104
S1 = 96
B = 4096
H = 100
C = 10

_info = plsc.get_sparse_core_info()
NC = _info.num_cores
NS = _info.num_subcores
NW = NC * NS
BPW = B // NW
NV = D // 16


def _bag_body(idx_hbm, table_hbm, out_hbm, idx_v, rows_v, out_v, sem0, sem1):
    wid = lax.axis_index("s") * NC + lax.axis_index("c")
    base = wid * BPW

    pltpu.sync_copy(idx_hbm.at[pl.ds(base, BPW)], idx_v)

    def issue(b, buf, sem):
        pltpu.async_copy(
            table_hbm.at[idx_v.at[b, pl.ds(0, S0)]],
            rows_v.at[buf, pl.ds(0, S0)], sem)
        pltpu.async_copy(
            table_hbm.at[idx_v.at[b, pl.ds(S0, S1)]],
            rows_v.at[buf, pl.ds(S0, S1)], sem)

    def drain(buf, sem):
        pltpu.make_async_copy(
            table_hbm.at[pl.ds(0, S0)], rows_v.at[buf, pl.ds(0, S0)],
            sem).wait()
        pltpu.make_async_copy(
            table_hbm.at[pl.ds(0, S1)], rows_v.at[buf, pl.ds(S0, S1)],
            sem).wait()

    def accumulate(b, buf):
        def row(r, accs):
            return tuple(
                accs[c] + rows_v[buf, r, pl.ds(c * 16, 16)]
                for c in range(NV)
            )

        zero = jnp.zeros((16,), jnp.float32)
        accs = lax.fori_loop(0, SEQ, row, (zero,) * NV)
        inv = jnp.float32(1.0 / SEQ)
        for c in range(NV):
            out_v[b, pl.ds(c * 16, 16)] = accs[c] * inv

    issue(0, 0, sem0)

    def body(i, _):
        b0 = 2 * i
        b1 = 2 * i + 1
        issue(b1, 1, sem1)
        drain(0, sem0)
        accumulate(b0, 0)

        @pl.when(b1 + 1 < BPW)
        def _():
            issue(b1 + 1, 0, sem0)

        drain(1, sem1)
        accumulate(b1, 1)
        return 0

    lax.fori_loop(0, BPW // 2, body, 0)
    pltpu.sync_copy(out_v, out_hbm.at[pl.ds(base, BPW)])


def _bag_means(idx, emb):
    mesh = plsc.VectorSubcoreMesh(core_axis_name="c", subcore_axis_name="s")
    f = functools.partial(
        pl.kernel,
        mesh=mesh,
        out_type=jax.ShapeDtypeStruct((B, D), jnp.float32),
        scratch_types=[
            pltpu.VMEM((BPW, SEQ), jnp.int32),
            pltpu.VMEM((2, SEQ, D), jnp.float32),
            pltpu.VMEM((BPW, D), jnp.float32),
            pltpu.SemaphoreType.DMA,
            pltpu.SemaphoreType.DMA,
        ],
        compiler_params=pltpu.CompilerParams(use_tc_tiling_on_sc=False),
    )(_bag_body)
    return f(idx, emb)


def _mlp_body(bag_ref, w1_ref, b1_ref, w2_ref, b2_ref, out_ref):
    h = jnp.dot(bag_ref[...], w1_ref[...], preferred_element_type=jnp.float32)
    h = jnp.maximum(h + b1_ref[...], 0.0)
    out_ref[...] = (
        jnp.dot(h, w2_ref[...], preferred_element_type=jnp.float32)
        + b2_ref[...]
    )


def _mlp(bag, W1, b1, W2, b2):
    return pl.pallas_call(
        _mlp_body,
        out_shape=jax.ShapeDtypeStruct((B, C), jnp.float32),
    )(bag, W1, b1.reshape(1, H), W2, b2.reshape(1, C))


def kernel(inputX, emb, W1, b1, W2, b2):
    idx = inputX.astype(jnp.int32)
    bag = _bag_means(idx, emb)
    return _mlp(bag, W1, b1, W2, b2)
